# baseline (speedup 1.0000x reference)
import functools

import jax
import jax.numpy as jnp
from jax import lax
from jax.experimental import pallas as pl
from jax.experimental.pallas import tpu as pltpu
from jax.experimental.pallas import tpu_sc as plsc

_F = 8
_N = 4096
_K = 16
_CS = 512
_CI = 64
_NB = 4096
_EPS = 1e-12
_MARGIN = 0.2
_HI = lax.Precision.HIGHEST
_DN = (((1,), (1,)), ((), ()))
_DNB = (((2,), (2,)), ((0,), (0,)))

_QPX = _N // 4
_CCH = 16
_NCHI = _CI // _CCH
_NPG = _QPX // 16


def _rsqrt16(x):
    i = plsc.bitcast(x, jnp.int32)
    y = plsc.bitcast(jnp.full((16,), 0x5F3759DF, jnp.int32) - (i >> 1),
                     jnp.float32)
    for _ in range(4):
        y = y * (1.5 - 0.5 * x * y * y)
    return y


def _sc_ins_body(fi_hbm, mi_hbm, mask_hbm, out_hbm,
                 ab0, sa0, ab1, sa1, mb0, sm0, mb1, sm1,
                 nab, nmb, damb, mmb, stg):
    wid = lax.axis_index("s") * 2 + lax.axis_index("c")
    f = wid // 4
    q = wid % 4
    p0 = q * _QPX
    abufs = ((ab0, sa0), (ab1, sa1))
    mbufs = ((mb0, sm0), (mb1, sm1))

    def _zero(i, _):
        s = pl.ds(i * 16, 16)
        z = jnp.zeros((16,), jnp.float32)
        nab[s] = z
        nmb[s] = z
        damb[s] = z
        return 0

    def _copies(g):
        ab, sa = abufs[g % 2]
        mb, sm = mbufs[g % 2]
        c0 = (g % _NCHI) * _CCH
        return (pltpu.make_async_copy(
                    fi_hbm.at[f, pl.ds(c0, _CCH), pl.ds(p0, _QPX)], ab, sa),
                pltpu.make_async_copy(
                    mi_hbm.at[f, pl.ds(c0, _CCH), pl.ds(p0, _QPX)], mb, sm))

    pltpu.sync_copy(mask_hbm.at[f, pl.ds(p0, _QPX)], mmb)
    lax.fori_loop(0, _NPG, _zero, 0)

    for c in _copies(0) + _copies(1):
        c.start()
    for g in range(_NCHI):
        for c in _copies(g):
            c.wait()
        ab, _ = abufs[g % 2]
        mb, _ = mbufs[g % 2]

        def _acc(pg, _):
            s = pl.ds(pg * 16, 16)
            na = nab[s]
            nm = nmb[s]
            dam = damb[s]
            for c in range(_CCH):
                va = ab[c, s]
                vm = mb[c, s]
                na = na + va * va
                nm = nm + vm * vm
                dam = dam + va * vm
            nab[s] = na
            nmb[s] = nm
            damb[s] = dam
            return 0

        lax.fori_loop(0, _NPG, _acc, 0)
        if g + 2 < _NCHI:
            for c in _copies(g + 2):
                c.start()

    def _cos_reduce(pg, acc):
        s = pl.ds(pg * 16, 16)
        inva = _rsqrt16(jnp.maximum(nab[s], 1e-24))
        invm = _rsqrt16(jnp.maximum(nmb[s], 1e-24))
        return acc + (1.0 - damb[s] * inva * invm) * mmb[s]

    cim = lax.fori_loop(0, _NPG, _cos_reduce, jnp.zeros((16,), jnp.float32))

    stg[pl.ds(0, 16)] = cim
    pltpu.sync_copy(stg, out_hbm.at[wid])


_sc_ins = functools.partial(
    pl.kernel,
    out_type=jax.ShapeDtypeStruct((32, 16), jnp.float32),
    mesh=plsc.VectorSubcoreMesh(core_axis_name="c", subcore_axis_name="s"),
    compiler_params=pltpu.CompilerParams(needs_layout_passes=False),
    scratch_types=[
        pltpu.VMEM((_CCH, _QPX), jnp.float32),
        pltpu.SemaphoreType.DMA,
        pltpu.VMEM((_CCH, _QPX), jnp.float32),
        pltpu.SemaphoreType.DMA,
        pltpu.VMEM((_CCH, _QPX), jnp.float32),
        pltpu.SemaphoreType.DMA,
        pltpu.VMEM((_CCH, _QPX), jnp.float32),
        pltpu.SemaphoreType.DMA,
        pltpu.VMEM((_QPX,), jnp.float32),
        pltpu.VMEM((_QPX,), jnp.float32),
        pltpu.VMEM((_QPX,), jnp.float32),
        pltpu.VMEM((_QPX,), jnp.float32),
        pltpu.VMEM((16,), jnp.float32),
    ],
)(_sc_ins_body)


def _tc_main_body(sem_ref, gt_ref, msem_ref, ins_ref, mask_ref, ids_ref,
                  sp_ref, sg_ref, ft_ref, cnt_ref, mm_ref, csm_ref):
    f = pl.program_id(0)

    @pl.when(f == 0)
    def _init():
        sp_ref[...] = jnp.zeros_like(sp_ref)
        sg_ref[...] = jnp.zeros_like(sg_ref)
        ft_ref[...] = jnp.zeros_like(ft_ref)
        cnt_ref[...] = jnp.zeros_like(cnt_ref)
        mm_ref[...] = jnp.zeros_like(mm_ref)
        csm_ref[...] = jnp.zeros_like(csm_ref)

    a = sem_ref[0]
    g = gt_ref[0]
    m = msem_ref[0]
    fi = ins_ref[0]
    mm = mask_ref[0]
    ids = ids_ref[0]

    na = jnp.sum(a * a, axis=0, keepdims=True)
    ng = jnp.sum(g * g, axis=0, keepdims=True)
    nm = jnp.sum(m * m, axis=0, keepdims=True)
    dam = jnp.sum(a * m, axis=0, keepdims=True)
    nfi = jnp.sum(fi * fi, axis=0, keepdims=True)
    inva = 1.0 / jnp.maximum(jnp.sqrt(na), _EPS)
    invg = 1.0 / jnp.maximum(jnp.sqrt(ng), _EPS)
    invm = 1.0 / jnp.maximum(jnp.sqrt(nm), _EPS)
    invf = 1.0 / jnp.maximum(jnp.sqrt(nfi), _EPS)

    mm_ref[...] += mm
    csm_ref[...] += (1.0 - dam * inva * invm) * mm

    oh = (ids == lax.broadcasted_iota(jnp.int32, (_K, _NB), 0)).astype(jnp.float32)

    sp_ref[f] += lax.dot_general(oh * inva, a, _DN,
                                 preferred_element_type=jnp.float32)
    sg_ref[f] += lax.dot_general(oh * invg, g, _DN,
                                 preferred_element_type=jnp.float32)
    ft_ref[f] += lax.dot_general(oh * invf, fi, _DN,
                                 preferred_element_type=jnp.float32)
    cnt_ref[f] += jnp.sum(oh, axis=1, keepdims=True)


def _finalize_body(sp_ref, sg_ref, ft_ref, cnt_ref, mm_ref, csm_ref,
                   sc_ref, out_ref):
    SP = sp_ref[...]
    SG = sg_ref[...]
    FT = ft_ref[...]
    cnt = cnt_ref[...]

    segk = lax.broadcasted_iota(jnp.int32, (_F, _K, 1), 1)
    fg = (segk > 0)

    ngp = jnp.sqrt(jnp.sum(SG * SG, axis=2, keepdims=True))
    dgp = jnp.sum(SG * SP, axis=2, keepdims=True)
    va = jnp.where(fg & (cnt >= 2.0), 1.0, 0.0)
    align_num = jnp.sum(va * (cnt - dgp / jnp.maximum(ngp, _EPS)))
    align_den = jnp.maximum(jnp.sum(va * cnt), 1.0)

    nf = jnp.sqrt(jnp.sum(FT * FT, axis=2, keepdims=True))
    vi = jnp.where(fg & (cnt >= 1.0), 1.0, 0.0)
    intra_num = jnp.sum(vi * (cnt - nf * nf / jnp.maximum(nf, _EPS)))
    intra_den = jnp.maximum(jnp.sum(vi * cnt), 1.0)

    pn = FT / jnp.maximum(nf, _EPS)
    sim = lax.dot_general(pn, pn, _DNB, precision=_HI,
                          preferred_element_type=jnp.float32)
    vv = lax.dot_general(vi, vi, _DNB, precision=_HI,
                         preferred_element_type=jnp.float32)
    r_i = lax.broadcasted_iota(jnp.int32, (_F, _K, _K), 1)
    c_i = lax.broadcasted_iota(jnp.int32, (_F, _K, _K), 2)
    pair = vv * jnp.where(r_i != c_i, 1.0, 0.0)
    inter_num = jnp.sum(jnp.maximum(sim - _MARGIN, 0.0) * pair)
    inter_den = jnp.maximum(jnp.sum(pair), 1.0)

    cim = jnp.sum(sc_ref[...])
    smm = jnp.maximum(jnp.sum(mm_ref[...]), 1.0)
    obj = (0.5 * align_num / align_den + jnp.sum(csm_ref[...]) / smm
           + intra_num / intra_den + inter_num / inter_den
           + cim / smm)
    out_ref[...] = obj[None, None]


def kernel(refined_sem, refined_ins, lseg_gt, mem_sem, mem_ins, mem_mask,
           inst_mask):
    sem = refined_sem.reshape(_F, _CS, _N)
    gt = lseg_gt.reshape(_F, _CS, _N)
    msem = mem_sem.reshape(_F, _CS, _N)
    ins = refined_ins.reshape(_F, _CI, _N)
    mins = mem_ins.reshape(_F, _CI, _N)
    mask2 = mem_mask.reshape(_F, _N)
    mask3 = mem_mask.reshape(_F, 1, _NB)
    ids3 = inst_mask.astype(jnp.int32).reshape(_F, 1, _NB)

    scp = _sc_ins(ins, mins, mask2)

    big_spec = pl.BlockSpec((1, _CS, _NB), lambda f: (f, 0, 0))
    ins_spec = pl.BlockSpec((1, _CI, _NB), lambda f: (f, 0, 0))
    row_spec = pl.BlockSpec((1, 1, _NB), lambda f: (f, 0, 0))

    def whole(shape):
        return pl.BlockSpec(shape, lambda f: tuple(0 for _ in shape))

    sp, sg, ft, cnt, mm, csm = pl.pallas_call(
        _tc_main_body,
        grid=(_F,),
        in_specs=[big_spec, big_spec, big_spec, ins_spec, row_spec,
                  row_spec],
        out_specs=[whole((_F, _K, _CS)), whole((_F, _K, _CS)),
                   whole((_F, _K, _CI)), whole((_F, _K, 1)),
                   whole((1, _NB)), whole((1, _NB))],
        out_shape=[jax.ShapeDtypeStruct((_F, _K, _CS), jnp.float32),
                   jax.ShapeDtypeStruct((_F, _K, _CS), jnp.float32),
                   jax.ShapeDtypeStruct((_F, _K, _CI), jnp.float32),
                   jax.ShapeDtypeStruct((_F, _K, 1), jnp.float32),
                   jax.ShapeDtypeStruct((1, _NB), jnp.float32),
                   jax.ShapeDtypeStruct((1, _NB), jnp.float32)],
    )(sem, gt, msem, ins, mask3, ids3)

    out = pl.pallas_call(
        _finalize_body,
        out_shape=jax.ShapeDtypeStruct((1, 1), jnp.float32),
    )(sp, sg, ft, cnt, mm, csm, scp)
    return out[0, 0]


# speedup vs baseline: 1.0079x; 1.0079x over previous
import functools

import jax
import jax.numpy as jnp
from jax import lax
from jax.experimental import pallas as pl
from jax.experimental.pallas import tpu as pltpu
from jax.experimental.pallas import tpu_sc as plsc

_F = 8
_N = 4096
_K = 16
_CS = 512
_CI = 64
_NB = 4096
_EPS = 1e-12
_MARGIN = 0.2
_HI = lax.Precision.HIGHEST
_DN = (((1,), (1,)), ((), ()))
_DNB = (((2,), (2,)), ((0,), (0,)))

_QPX = _N // 4
_CCH = 16
_NCHI = _CI // _CCH
_NPG = _QPX // 16


def _rsqrt16(x):
    i = plsc.bitcast(x, jnp.int32)
    y = plsc.bitcast(jnp.full((16,), 0x5F3759DF, jnp.int32) - (i >> 1),
                     jnp.float32)
    for _ in range(4):
        y = y * (1.5 - 0.5 * x * y * y)
    return y


def _sc_ins_body(fi_hbm, mi_hbm, mask_hbm, out_hbm,
                 ab0, sa0, ab1, sa1, mb0, sm0, mb1, sm1,
                 nab, nmb, damb, mmb, stg):
    wid = lax.axis_index("s") * 2 + lax.axis_index("c")
    f = wid // 4
    q = wid % 4
    p0 = q * _QPX
    abufs = ((ab0, sa0), (ab1, sa1))
    mbufs = ((mb0, sm0), (mb1, sm1))

    def _zero(i, _):
        s = pl.ds(i * 16, 16)
        z = jnp.zeros((16,), jnp.float32)
        nab[s] = z
        nmb[s] = z
        damb[s] = z
        return 0

    def _copies(g):
        ab, sa = abufs[g % 2]
        mb, sm = mbufs[g % 2]
        c0 = (g % _NCHI) * _CCH
        return (pltpu.make_async_copy(
                    fi_hbm.at[f, pl.ds(c0, _CCH), pl.ds(p0, _QPX)], ab, sa),
                pltpu.make_async_copy(
                    mi_hbm.at[f, pl.ds(c0, _CCH), pl.ds(p0, _QPX)], mb, sm))

    pltpu.sync_copy(mask_hbm.at[f, pl.ds(p0, _QPX)], mmb)
    lax.fori_loop(0, _NPG, _zero, 0)

    for c in _copies(0) + _copies(1):
        c.start()
    for g in range(_NCHI):
        for c in _copies(g):
            c.wait()
        ab, _ = abufs[g % 2]
        mb, _ = mbufs[g % 2]

        def _acc(pg, _):
            s = pl.ds(pg * 16, 16)
            na = nab[s]
            nm = nmb[s]
            dam = damb[s]
            for c in range(_CCH):
                va = ab[c, s]
                vm = mb[c, s]
                na = na + va * va
                nm = nm + vm * vm
                dam = dam + va * vm
            nab[s] = na
            nmb[s] = nm
            damb[s] = dam
            return 0

        lax.fori_loop(0, _NPG, _acc, 0)
        if g + 2 < _NCHI:
            for c in _copies(g + 2):
                c.start()

    def _cos_reduce(pg, acc):
        s = pl.ds(pg * 16, 16)
        inva = _rsqrt16(jnp.maximum(nab[s], 1e-24))
        invm = _rsqrt16(jnp.maximum(nmb[s], 1e-24))
        return acc + (1.0 - damb[s] * inva * invm) * mmb[s]

    cim = lax.fori_loop(0, _NPG, _cos_reduce, jnp.zeros((16,), jnp.float32))

    stg[pl.ds(0, 16)] = cim
    pltpu.sync_copy(stg, out_hbm.at[wid])


_sc_ins = functools.partial(
    pl.kernel,
    out_type=jax.ShapeDtypeStruct((32, 16), jnp.float32),
    mesh=plsc.VectorSubcoreMesh(core_axis_name="c", subcore_axis_name="s"),
    compiler_params=pltpu.CompilerParams(needs_layout_passes=False),
    scratch_types=[
        pltpu.VMEM((_CCH, _QPX), jnp.float32),
        pltpu.SemaphoreType.DMA,
        pltpu.VMEM((_CCH, _QPX), jnp.float32),
        pltpu.SemaphoreType.DMA,
        pltpu.VMEM((_CCH, _QPX), jnp.float32),
        pltpu.SemaphoreType.DMA,
        pltpu.VMEM((_CCH, _QPX), jnp.float32),
        pltpu.SemaphoreType.DMA,
        pltpu.VMEM((_QPX,), jnp.float32),
        pltpu.VMEM((_QPX,), jnp.float32),
        pltpu.VMEM((_QPX,), jnp.float32),
        pltpu.VMEM((_QPX,), jnp.float32),
        pltpu.VMEM((16,), jnp.float32),
    ],
)(_sc_ins_body)


def _tc_main_body(sem_ref, gt_ref, msem_ref, ins_ref, mask_ref, ids_ref,
                  scp_ref, out_ref, sp_ref, sg_ref, ft_ref, cnt_ref,
                  mm_ref, csm_ref):
    f = pl.program_id(0)

    @pl.when(f == 0)
    def _init():
        sp_ref[...] = jnp.zeros_like(sp_ref)
        sg_ref[...] = jnp.zeros_like(sg_ref)
        ft_ref[...] = jnp.zeros_like(ft_ref)
        cnt_ref[...] = jnp.zeros_like(cnt_ref)
        mm_ref[...] = jnp.zeros_like(mm_ref)
        csm_ref[...] = jnp.zeros_like(csm_ref)

    a = sem_ref[0]
    g = gt_ref[0]
    m = msem_ref[0]
    fi = ins_ref[0]
    mm = mask_ref[0]
    ids = ids_ref[0]

    na = jnp.sum(a * a, axis=0, keepdims=True)
    ng = jnp.sum(g * g, axis=0, keepdims=True)
    nm = jnp.sum(m * m, axis=0, keepdims=True)
    dam = jnp.sum(a * m, axis=0, keepdims=True)
    nfi = jnp.sum(fi * fi, axis=0, keepdims=True)
    inva = 1.0 / jnp.maximum(jnp.sqrt(na), _EPS)
    invg = 1.0 / jnp.maximum(jnp.sqrt(ng), _EPS)
    invm = 1.0 / jnp.maximum(jnp.sqrt(nm), _EPS)
    invf = 1.0 / jnp.maximum(jnp.sqrt(nfi), _EPS)

    mm_ref[...] += mm
    csm_ref[...] += (1.0 - dam * inva * invm) * mm

    oh = (ids == lax.broadcasted_iota(jnp.int32, (_K, _NB), 0)).astype(jnp.float32)

    sp_ref[f] += lax.dot_general(oh * inva, a, _DN,
                                 preferred_element_type=jnp.float32)
    sg_ref[f] += lax.dot_general(oh * invg, g, _DN,
                                 preferred_element_type=jnp.float32)
    ft_ref[f] += lax.dot_general(oh * invf, fi, _DN,
                                 preferred_element_type=jnp.float32)
    cnt_ref[f] += jnp.sum(oh, axis=1, keepdims=True)

    @pl.when(f == _F - 1)
    def _finalize():
        _finalize_math(sp_ref, sg_ref, ft_ref, cnt_ref, mm_ref, csm_ref,
                       scp_ref, out_ref)


def _finalize_math(sp_ref, sg_ref, ft_ref, cnt_ref, mm_ref, csm_ref,
                   sc_ref, out_ref):
    SP = sp_ref[...]
    SG = sg_ref[...]
    FT = ft_ref[...]
    cnt = cnt_ref[...]

    segk = lax.broadcasted_iota(jnp.int32, (_F, _K, 1), 1)
    fg = (segk > 0)

    ngp = jnp.sqrt(jnp.sum(SG * SG, axis=2, keepdims=True))
    dgp = jnp.sum(SG * SP, axis=2, keepdims=True)
    va = jnp.where(fg & (cnt >= 2.0), 1.0, 0.0)
    align_num = jnp.sum(va * (cnt - dgp / jnp.maximum(ngp, _EPS)))
    align_den = jnp.maximum(jnp.sum(va * cnt), 1.0)

    nf = jnp.sqrt(jnp.sum(FT * FT, axis=2, keepdims=True))
    vi = jnp.where(fg & (cnt >= 1.0), 1.0, 0.0)
    intra_num = jnp.sum(vi * (cnt - nf * nf / jnp.maximum(nf, _EPS)))
    intra_den = jnp.maximum(jnp.sum(vi * cnt), 1.0)

    pn = FT / jnp.maximum(nf, _EPS)
    sim = lax.dot_general(pn, pn, _DNB, precision=_HI,
                          preferred_element_type=jnp.float32)
    vv = lax.dot_general(vi, vi, _DNB, precision=_HI,
                         preferred_element_type=jnp.float32)
    r_i = lax.broadcasted_iota(jnp.int32, (_F, _K, _K), 1)
    c_i = lax.broadcasted_iota(jnp.int32, (_F, _K, _K), 2)
    pair = vv * jnp.where(r_i != c_i, 1.0, 0.0)
    inter_num = jnp.sum(jnp.maximum(sim - _MARGIN, 0.0) * pair)
    inter_den = jnp.maximum(jnp.sum(pair), 1.0)

    cim = jnp.sum(sc_ref[...])
    smm = jnp.maximum(jnp.sum(mm_ref[...]), 1.0)
    obj = (0.5 * align_num / align_den + jnp.sum(csm_ref[...]) / smm
           + intra_num / intra_den + inter_num / inter_den
           + cim / smm)
    out_ref[...] = obj[None, None]


def kernel(refined_sem, refined_ins, lseg_gt, mem_sem, mem_ins, mem_mask,
           inst_mask):
    sem = refined_sem.reshape(_F, _CS, _N)
    gt = lseg_gt.reshape(_F, _CS, _N)
    msem = mem_sem.reshape(_F, _CS, _N)
    ins = refined_ins.reshape(_F, _CI, _N)
    mins = mem_ins.reshape(_F, _CI, _N)
    mask2 = mem_mask.reshape(_F, _N)
    mask3 = mem_mask.reshape(_F, 1, _NB)
    ids3 = inst_mask.astype(jnp.int32).reshape(_F, 1, _NB)

    scp = _sc_ins(ins, mins, mask2)

    big_spec = pl.BlockSpec((1, _CS, _NB), lambda f: (f, 0, 0))
    ins_spec = pl.BlockSpec((1, _CI, _NB), lambda f: (f, 0, 0))
    row_spec = pl.BlockSpec((1, 1, _NB), lambda f: (f, 0, 0))

    scp_spec = pl.BlockSpec((32, _K), lambda f: (0, 0))

    out = pl.pallas_call(
        _tc_main_body,
        grid=(_F,),
        in_specs=[big_spec, big_spec, big_spec, ins_spec, row_spec,
                  row_spec, scp_spec],
        out_specs=pl.BlockSpec((1, 1), lambda f: (0, 0)),
        out_shape=jax.ShapeDtypeStruct((1, 1), jnp.float32),
        scratch_shapes=[
            pltpu.VMEM((_F, _K, _CS), jnp.float32),
            pltpu.VMEM((_F, _K, _CS), jnp.float32),
            pltpu.VMEM((_F, _K, _CI), jnp.float32),
            pltpu.VMEM((_F, _K, 1), jnp.float32),
            pltpu.VMEM((1, _NB), jnp.float32),
            pltpu.VMEM((1, _NB), jnp.float32),
        ],
    )(sem, gt, msem, ins, mask3, ids3, scp)
    return out[0, 0]
